# baseline (speedup 1.0000x reference)
import functools

import jax
import jax.numpy as jnp
from jax import lax
from jax.experimental import pallas as pl
from jax.experimental.pallas import tpu as pltpu
from jax.experimental.pallas import tpu_sc as plsc

D = 128
N_TO = 10000
LANES = 128
NC, NS = 2, 16
NW = NC * NS
ACC_ROWS = 10240
ZROWS = ACC_ROWS // NS


def _sc_partials(x, edges, n_chunks):
    mesh = plsc.VectorSubcoreMesh(core_axis_name="c", subcore_axis_name="s")

    nfull = n_chunks // NW
    nrem = n_chunks - nfull * NW
    h0 = nfull // 2
    if h0 % 2:
        h0 += 1
    h1 = nfull - h0
    hbuf = max(h0, 1)

    @functools.partial(
        pl.kernel,
        out_type=jax.ShapeDtypeStruct((NC, ACC_ROWS, D), jnp.float32),
        mesh=mesh,
        scratch_types=[
            pltpu.VMEM((2, hbuf * LANES), jnp.int32),
            pltpu.VMEM((LANES, D), jnp.float32),
            pltpu.VMEM((LANES, D), jnp.float32),
            pltpu.VMEM_SHARED((ACC_ROWS, D), jnp.float32),
            pltpu.SemaphoreType.DMA,
            pltpu.SemaphoreType.DMA,
            pltpu.SemaphoreType.DMA,
            pltpu.SemaphoreType.DMA,
            pltpu.SemaphoreType.DMA,
        ],
    )
    def k(x_hbm, e_hbm, outp_hbm,
          idx_v, rows0, rows1, acc_sh,
          gsem0, gsem1, ssem0, ssem1, zsem):
        c = lax.axis_index("c")
        s = lax.axis_index("s")
        w = c * NS + s
        base_e = w * (nfull * LANES)
        rows = (rows0, rows1)
        gsem = (gsem0, gsem1)
        ssem = (ssem0, ssem1)

        zv = jnp.zeros((16,), jnp.float32)

        @pl.loop(0, LANES)
        def _zrow(i):
            for kk in range(D // 16):
                rows0[i, pl.ds(kk * 16, 16)] = zv

        nz = ZROWS // LANES
        for zi in range(nz):
            pltpu.async_copy(
                rows0, acc_sh.at[pl.ds(s * ZROWS + zi * LANES, LANES)], zsem)

        def g_start(j, b):
            pltpu.async_copy(
                x_hbm.at[idx_v.at[0, pl.ds(j * LANES, LANES)]],
                rows[b], gsem[b])

        def g_wait(b):
            pltpu.make_async_copy(
                x_hbm.at[idx_v.at[0, pl.ds(0, LANES)]],
                rows[b], gsem[b]).wait()

        def s_start(j, b):
            pltpu.async_copy(
                rows[b], acc_sh.at[idx_v.at[1, pl.ds(j * LANES, LANES)]],
                ssem[b], add=True)

        def s_wait(b):
            pltpu.make_async_copy(
                rows[b], acc_sh.at[idx_v.at[1, pl.ds(0, LANES)]],
                ssem[b]).wait()

        first = True
        for off, h in ((0, h0), (h0, h1)):
            ne = h * LANES
            pltpu.sync_copy(
                e_hbm.at[:, pl.ds(base_e + off * LANES, ne)],
                idx_v.at[:, pl.ds(0, ne)])
            if first:
                first = False
                for zi in range(nz):
                    pltpu.make_async_copy(
                        rows0, acc_sh.at[pl.ds(s * ZROWS, LANES)],
                        zsem).wait()
                plsc.subcore_barrier()

            g_start(0, 0)
            g_wait(0)
            g_start(1, 1)
            s_start(0, 0)
            g_wait(1)
            s_wait(0)
            g_start(2, 0)
            s_start(1, 1)

            @pl.loop(2, h - 2, step=2)
            def _pipeline(j):
                g_wait(0)
                s_wait(1)
                g_start(j + 1, 1)
                s_start(j, 0)
                g_wait(1)
                s_wait(0)
                g_start(j + 2, 0)
                s_start(j + 1, 1)

            g_wait(0)
            s_wait(1)
            g_start(h - 1, 1)
            s_start(h - 2, 0)
            g_wait(1)
            s_wait(0)
            s_start(h - 1, 1)
            s_wait(1)

        if nrem:
            r = s * NC + c

            @pl.when(r < nrem)
            def _rem():
                rbase = (n_chunks - nrem) * LANES + r * LANES
                pltpu.sync_copy(e_hbm.at[:, pl.ds(rbase, LANES)],
                                idx_v.at[:, pl.ds(0, LANES)])
                g_start(0, 0)
                g_wait(0)
                s_start(0, 0)
                s_wait(0)

        plsc.subcore_barrier()

        pltpu.sync_copy(
            acc_sh.at[pl.ds(s * ZROWS, ZROWS)],
            outp_hbm.at[c, pl.ds(s * ZROWS, ZROWS)],
        )

    return k(x, edges)


def _combine_body(a_ref, b_ref, o_ref):
    o_ref[...] = a_ref[0] + b_ref[0]


def kernel(x, edge_index, num_nodes_to):
    del num_nodes_to
    e = edge_index.shape[1]
    edges = edge_index.astype(jnp.int32)

    if e % LANES:
        npad = LANES - e % LANES
        pad_ar = jnp.arange(npad, dtype=jnp.int32)
        pad = jnp.stack([pad_ar % x.shape[0],
                         N_TO + pad_ar % (ACC_ROWS - N_TO)])
        edges = jnp.concatenate([edges, pad], axis=1)
    n_chunks = edges.shape[1] // LANES

    partials = _sc_partials(x, edges, n_chunks)

    rows_per_blk = 1000
    out = pl.pallas_call(
        _combine_body,
        out_shape=jax.ShapeDtypeStruct((N_TO, D), jnp.float32),
        grid=(N_TO // rows_per_blk,),
        in_specs=[
            pl.BlockSpec((1, rows_per_blk, D), lambda i: (0, i, 0)),
            pl.BlockSpec((1, rows_per_blk, D), lambda i: (1, i, 0)),
        ],
        out_specs=pl.BlockSpec((rows_per_blk, D), lambda i: (i, 0)),
    )(partials, partials)
    return out


# speedup vs baseline: 1.2852x; 1.2852x over previous
import functools

import jax
import jax.numpy as jnp
from jax import lax
from jax.experimental import pallas as pl
from jax.experimental.pallas import tpu as pltpu
from jax.experimental.pallas import tpu_sc as plsc

D = 128
N_TO = 10000
LANES = 64
NBUF = 4
NC, NS = 2, 16
NW = NC * NS
ACC_ROWS = 10240
ZROWS = ACC_ROWS // NS


def _sc_partials(x, edges, n_chunks):
    mesh = plsc.VectorSubcoreMesh(core_axis_name="c", subcore_axis_name="s")

    nfull = n_chunks // NW
    nrem = n_chunks - nfull * NW
    h0 = nfull // 2
    h0 += (-h0) % 4
    h1 = nfull - h0
    hbuf = max(h0, 2)

    @functools.partial(
        pl.kernel,
        out_type=jax.ShapeDtypeStruct((NC, ACC_ROWS, D), jnp.float32),
        mesh=mesh,
        scratch_types=[
            pltpu.VMEM((2, hbuf * LANES), jnp.int32),
            [pltpu.VMEM((LANES, D), jnp.float32)] * NBUF,
            pltpu.VMEM_SHARED((ACC_ROWS, D), jnp.float32),
            [pltpu.SemaphoreType.DMA] * NBUF,
            [pltpu.SemaphoreType.DMA] * NBUF,
            pltpu.SemaphoreType.DMA,
        ],
    )
    def k(x_hbm, e_hbm, outp_hbm, idx_v, rows, acc_sh, gsem, ssem, zsem):
        c = lax.axis_index("c")
        s = lax.axis_index("s")
        w = c * NS + s
        base_e = w * (nfull * LANES)

        zv = jnp.zeros((16,), jnp.float32)

        @pl.loop(0, LANES)
        def _zrow(i):
            for kk in range(D // 16):
                rows[0][i, pl.ds(kk * 16, 16)] = zv

        nz = ZROWS // LANES
        for zi in range(nz):
            pltpu.async_copy(
                rows[0], acc_sh.at[pl.ds(s * ZROWS + zi * LANES, LANES)],
                zsem)

        def g_start(j, b):
            pltpu.async_copy(
                x_hbm.at[idx_v.at[0, pl.ds(j * LANES, LANES)]],
                rows[b], gsem[b])

        def g_wait(b):
            pltpu.make_async_copy(
                x_hbm.at[idx_v.at[0, pl.ds(0, LANES)]],
                rows[b], gsem[b]).wait()

        def s_start(j, b):
            pltpu.async_copy(
                rows[b], acc_sh.at[idx_v.at[1, pl.ds(j * LANES, LANES)]],
                ssem[b], add=True)

        def s_wait(b):
            pltpu.make_async_copy(
                rows[b], acc_sh.at[idx_v.at[1, pl.ds(0, LANES)]],
                ssem[b]).wait()

        first = True
        for off, h in ((0, h0), (h0, h1)):
            ne = h * LANES
            pltpu.sync_copy(
                e_hbm.at[:, pl.ds(base_e + off * LANES, ne)],
                idx_v.at[:, pl.ds(0, ne)])
            if first:
                first = False
                for zi in range(nz):
                    pltpu.make_async_copy(
                        rows[0], acc_sh.at[pl.ds(s * ZROWS, LANES)],
                        zsem).wait()
                plsc.subcore_barrier()

            g_start(0, 0)
            g_start(1, 1)
            g_start(2, 2)
            g_wait(0)
            g_start(3, 3)
            s_start(0, 0)
            g_wait(1)
            s_wait(0)
            g_start(4, 0)
            s_start(1, 1)
            g_wait(2)
            s_wait(1)
            g_start(5, 1)
            s_start(2, 2)
            g_wait(3)
            s_wait(2)
            g_start(6, 2)
            s_start(3, 3)

            @pl.loop(4, h - 4, step=4)
            def _pipeline(j):
                for u in range(4):
                    b = u
                    g_wait(b)
                    s_wait((b + 3) % 4)
                    g_start(j + u + 3, (b + 3) % 4)
                    s_start(j + u, b)

            g_wait(0)
            s_wait(3)
            g_start(h - 1, 3)
            s_start(h - 4, 0)
            for u in (1, 2, 3):
                g_wait(u)
                s_wait(u - 1)
                s_start(h - 4 + u, u)
            s_wait(3)

        if nrem:
            npair = nrem // 2
            r = s * NC + c

            @pl.when(r < npair)
            def _rem():
                rbase = (n_chunks - nrem) * LANES + r * (2 * LANES)
                pltpu.sync_copy(e_hbm.at[:, pl.ds(rbase, 2 * LANES)],
                                idx_v.at[:, pl.ds(0, 2 * LANES)])
                g_start(0, 0)
                g_start(1, 1)
                g_wait(0)
                s_start(0, 0)
                g_wait(1)
                s_start(1, 1)
                s_wait(0)
                s_wait(1)

        plsc.subcore_barrier()

        pltpu.sync_copy(
            acc_sh.at[pl.ds(s * ZROWS, ZROWS)],
            outp_hbm.at[c, pl.ds(s * ZROWS, ZROWS)],
        )

    return k(x, edges)


def _combine_body(a_ref, b_ref, o_ref):
    o_ref[...] = a_ref[0] + b_ref[0]


def kernel(x, edge_index, num_nodes_to):
    del num_nodes_to
    e = edge_index.shape[1]
    edges = edge_index.astype(jnp.int32)

    if e % (2 * LANES):
        npad = (2 * LANES) - e % (2 * LANES)
        pad_ar = jnp.arange(npad, dtype=jnp.int32)
        pad = jnp.stack([pad_ar % x.shape[0],
                         N_TO + pad_ar % (ACC_ROWS - N_TO)])
        edges = jnp.concatenate([edges, pad], axis=1)
    n_chunks = edges.shape[1] // LANES

    partials = _sc_partials(x, edges, n_chunks)

    rows_per_blk = 1000
    out = pl.pallas_call(
        _combine_body,
        out_shape=jax.ShapeDtypeStruct((N_TO, D), jnp.float32),
        grid=(N_TO // rows_per_blk,),
        in_specs=[
            pl.BlockSpec((1, rows_per_blk, D), lambda i: (0, i, 0)),
            pl.BlockSpec((1, rows_per_blk, D), lambda i: (1, i, 0)),
        ],
        out_specs=pl.BlockSpec((rows_per_blk, D), lambda i: (i, 0)),
    )(partials, partials)
    return out


# speedup vs baseline: 1.2986x; 1.0104x over previous
import functools

import jax
import jax.numpy as jnp
from jax import lax
from jax.experimental import pallas as pl
from jax.experimental.pallas import tpu as pltpu
from jax.experimental.pallas import tpu_sc as plsc

D = 128
N_TO = 10000
LANES = 64
NBUF = 4
NC, NS = 2, 16
NW = NC * NS
ACC_ROWS = 10240
ZROWS = ACC_ROWS // NS


def _sc_partials(x, edges, n_chunks):
    mesh = plsc.VectorSubcoreMesh(core_axis_name="c", subcore_axis_name="s")

    nfull = n_chunks // NW
    nrem = n_chunks - nfull * NW
    h0 = nfull // 2
    h0 += (-h0) % 4
    h1 = nfull - h0
    hbuf = max(h0, 2)

    @functools.partial(
        pl.kernel,
        out_type=jax.ShapeDtypeStruct((NC, ACC_ROWS, D), jnp.float32),
        mesh=mesh,
        scratch_types=[
            pltpu.VMEM((2, hbuf * LANES), jnp.int32),
            [pltpu.VMEM((LANES, D), jnp.float32)] * NBUF,
            pltpu.VMEM_SHARED((ACC_ROWS, D), jnp.float32),
            [pltpu.SemaphoreType.DMA] * NBUF,
            [pltpu.SemaphoreType.DMA] * NBUF,
            pltpu.SemaphoreType.DMA,
            pltpu.SemaphoreType.DMA,
        ],
    )
    def k(x_hbm, e_hbm, outp_hbm, idx_v, rows, acc_sh, gsem, ssem, zsem,
          isem):
        c = lax.axis_index("c")
        s = lax.axis_index("s")
        w = c * NS + s
        base_e = w * (nfull * LANES)

        ne0 = h0 * LANES
        icopy = pltpu.async_copy(
            e_hbm.at[:, pl.ds(base_e, ne0)],
            idx_v.at[:, pl.ds(0, ne0)], isem)

        zv = jnp.zeros((16,), jnp.float32)
        zbuf = rows[NBUF - 1]

        @pl.loop(0, LANES)
        def _zrow(i):
            for kk in range(D // 16):
                zbuf[i, pl.ds(kk * 16, 16)] = zv

        nz = ZROWS // LANES
        for zi in range(nz):
            pltpu.async_copy(
                zbuf, acc_sh.at[pl.ds(s * ZROWS + zi * LANES, LANES)],
                zsem)

        def g_start(j, b):
            pltpu.async_copy(
                x_hbm.at[idx_v.at[0, pl.ds(j * LANES, LANES)]],
                rows[b], gsem[b])

        def g_wait(b):
            pltpu.make_async_copy(
                x_hbm.at[idx_v.at[0, pl.ds(0, LANES)]],
                rows[b], gsem[b]).wait()

        def s_start(j, b):
            pltpu.async_copy(
                rows[b], acc_sh.at[idx_v.at[1, pl.ds(j * LANES, LANES)]],
                ssem[b], add=True)

        def s_wait(b):
            pltpu.make_async_copy(
                rows[b], acc_sh.at[idx_v.at[1, pl.ds(0, LANES)]],
                ssem[b]).wait()

        first = True
        for off, h in ((0, h0), (h0, h1)):
            if first:
                first = False
                icopy.wait()
                g_start(0, 0)
                g_start(1, 1)
                g_start(2, 2)
                for zi in range(nz):
                    pltpu.make_async_copy(
                        zbuf, acc_sh.at[pl.ds(s * ZROWS, LANES)],
                        zsem).wait()
                plsc.subcore_barrier()
            else:
                ne = h * LANES
                pltpu.sync_copy(
                    e_hbm.at[:, pl.ds(base_e + off * LANES, ne)],
                    idx_v.at[:, pl.ds(0, ne)])
                g_start(0, 0)
                g_start(1, 1)
                g_start(2, 2)

            g_wait(0)
            g_start(3, 3)
            s_start(0, 0)
            g_wait(1)
            s_wait(0)
            g_start(4, 0)
            s_start(1, 1)
            g_wait(2)
            s_wait(1)
            g_start(5, 1)
            s_start(2, 2)
            g_wait(3)
            s_wait(2)
            g_start(6, 2)
            s_start(3, 3)

            @pl.loop(4, h - 4, step=4)
            def _pipeline(j):
                for u in range(4):
                    b = u
                    g_wait(b)
                    s_wait((b + 3) % 4)
                    g_start(j + u + 3, (b + 3) % 4)
                    s_start(j + u, b)

            g_wait(0)
            s_wait(3)
            g_start(h - 1, 3)
            s_start(h - 4, 0)
            for u in (1, 2, 3):
                g_wait(u)
                s_wait(u - 1)
                s_start(h - 4 + u, u)
            s_wait(3)

        if nrem:
            npair = nrem // 2
            r = s * NC + c

            @pl.when(r < npair)
            def _rem():
                rbase = (n_chunks - nrem) * LANES + r * (2 * LANES)
                pltpu.sync_copy(e_hbm.at[:, pl.ds(rbase, 2 * LANES)],
                                idx_v.at[:, pl.ds(0, 2 * LANES)])
                g_start(0, 0)
                g_start(1, 1)
                g_wait(0)
                s_start(0, 0)
                g_wait(1)
                s_start(1, 1)
                s_wait(0)
                s_wait(1)

        plsc.subcore_barrier()

        pltpu.sync_copy(
            acc_sh.at[pl.ds(s * ZROWS, ZROWS)],
            outp_hbm.at[c, pl.ds(s * ZROWS, ZROWS)],
        )

    return k(x, edges)


def _combine_body(a_ref, b_ref, o_ref):
    o_ref[...] = a_ref[0] + b_ref[0]


def kernel(x, edge_index, num_nodes_to):
    del num_nodes_to
    e = edge_index.shape[1]
    edges = edge_index.astype(jnp.int32)

    if e % (2 * LANES):
        npad = (2 * LANES) - e % (2 * LANES)
        pad_ar = jnp.arange(npad, dtype=jnp.int32)
        pad = jnp.stack([pad_ar % x.shape[0],
                         N_TO + pad_ar % (ACC_ROWS - N_TO)])
        edges = jnp.concatenate([edges, pad], axis=1)
    n_chunks = edges.shape[1] // LANES

    partials = _sc_partials(x, edges, n_chunks)

    rows_per_blk = 1000
    out = pl.pallas_call(
        _combine_body,
        out_shape=jax.ShapeDtypeStruct((N_TO, D), jnp.float32),
        grid=(N_TO // rows_per_blk,),
        in_specs=[
            pl.BlockSpec((1, rows_per_blk, D), lambda i: (0, i, 0)),
            pl.BlockSpec((1, rows_per_blk, D), lambda i: (1, i, 0)),
        ],
        out_specs=pl.BlockSpec((rows_per_blk, D), lambda i: (i, 0)),
    )(partials, partials)
    return out


# speedup vs baseline: 1.3252x; 1.0205x over previous
import functools

import jax
import jax.numpy as jnp
from jax import lax
from jax.experimental import pallas as pl
from jax.experimental.pallas import tpu as pltpu
from jax.experimental.pallas import tpu_sc as plsc

D = 128
N_TO = 10000
LANES = 64
NBUF = 4
NC, NS = 2, 16
NW = NC * NS
ACC_ROWS = 10240
ZROWS = ACC_ROWS // NS


def _sc_partials(x, edges, n_chunks):
    mesh = plsc.VectorSubcoreMesh(core_axis_name="c", subcore_axis_name="s")

    nfull = n_chunks // NW
    nrem = n_chunks - nfull * NW
    h0 = nfull // 2
    h0 += (-h0) % 4
    h1 = nfull - h0
    hbuf = max(h0, 2)

    @functools.partial(
        pl.kernel,
        out_type=jax.ShapeDtypeStruct((NC, ACC_ROWS, D), jnp.float32),
        mesh=mesh,
        scratch_types=[
            pltpu.VMEM((2, hbuf * LANES), jnp.int32),
            [pltpu.VMEM((LANES, D), jnp.float32)] * NBUF,
            pltpu.VMEM_SHARED((ACC_ROWS, D), jnp.float32),
            [pltpu.SemaphoreType.DMA] * NBUF,
            [pltpu.SemaphoreType.DMA] * NBUF,
            pltpu.SemaphoreType.DMA,
            pltpu.SemaphoreType.DMA,
        ],
    )
    def k(x_hbm, e_hbm, outp_hbm, idx_v, rows, acc_sh, gsem, ssem, zsem,
          isem):
        c = lax.axis_index("c")
        s = lax.axis_index("s")
        w = c * NS + s
        base_e = w * (nfull * LANES)

        ne0 = h0 * LANES
        icopy = pltpu.async_copy(
            e_hbm.at[:, pl.ds(base_e, ne0)],
            idx_v.at[:, pl.ds(0, ne0)], isem)

        zv = jnp.zeros((16,), jnp.float32)
        zbuf = rows[NBUF - 1]

        @pl.loop(0, LANES)
        def _zrow(i):
            for kk in range(D // 16):
                zbuf[i, pl.ds(kk * 16, 16)] = zv

        nz = ZROWS // LANES
        for zi in range(nz):
            pltpu.async_copy(
                zbuf, acc_sh.at[pl.ds(s * ZROWS + zi * LANES, LANES)],
                zsem)

        def g_start(j, b):
            pltpu.async_copy(
                x_hbm.at[idx_v.at[0, pl.ds(j * LANES, LANES)]],
                rows[b], gsem[b])

        def g_wait(b):
            pltpu.make_async_copy(
                x_hbm.at[idx_v.at[0, pl.ds(0, LANES)]],
                rows[b], gsem[b]).wait()

        def s_start(j, b):
            pltpu.async_copy(
                rows[b], acc_sh.at[idx_v.at[1, pl.ds(j * LANES, LANES)]],
                ssem[b], add=True)

        def s_wait(b):
            pltpu.make_async_copy(
                rows[b], acc_sh.at[idx_v.at[1, pl.ds(0, LANES)]],
                ssem[b]).wait()

        first = True
        for off, h in ((0, h0), (h0, h1)):
            if first:
                first = False
                icopy.wait()
                g_start(0, 0)
                g_start(1, 1)
                g_start(2, 2)
                for zi in range(nz):
                    pltpu.make_async_copy(
                        zbuf, acc_sh.at[pl.ds(s * ZROWS, LANES)],
                        zsem).wait()
                plsc.subcore_barrier()
            else:
                ne = h * LANES
                pltpu.sync_copy(
                    e_hbm.at[:, pl.ds(base_e + off * LANES, ne)],
                    idx_v.at[:, pl.ds(0, ne)])
                g_start(0, 0)
                g_start(1, 1)
                g_start(2, 2)

            g_wait(0)
            g_start(3, 3)
            s_start(0, 0)
            g_wait(1)
            s_wait(0)
            g_start(4, 0)
            s_start(1, 1)
            g_wait(2)
            s_wait(1)
            g_start(5, 1)
            s_start(2, 2)
            g_wait(3)
            s_wait(2)
            g_start(6, 2)
            s_start(3, 3)

            @pl.loop(4, h - 4, step=4)
            def _pipeline(j):
                for u in range(4):
                    b = u
                    g_wait(b)
                    s_wait((b + 3) % 4)
                    g_start(j + u + 3, (b + 3) % 4)
                    s_start(j + u, b)

            g_wait(0)
            s_wait(3)
            g_start(h - 1, 3)
            s_start(h - 4, 0)
            for u in (1, 2, 3):
                g_wait(u)
                s_wait(u - 1)
                s_start(h - 4 + u, u)
            s_wait(3)

        if nrem:
            npair = nrem // 2
            r = s * NC + c

            @pl.when(r < npair)
            def _rem():
                rbase = (n_chunks - nrem) * LANES + r * (2 * LANES)
                pltpu.sync_copy(e_hbm.at[:, pl.ds(rbase, 2 * LANES)],
                                idx_v.at[:, pl.ds(0, 2 * LANES)])
                g_start(0, 0)
                g_start(1, 1)
                g_wait(0)
                s_start(0, 0)
                g_wait(1)
                s_start(1, 1)
                s_wait(0)
                s_wait(1)

        plsc.subcore_barrier()

        pltpu.sync_copy(
            acc_sh.at[pl.ds(s * ZROWS, ZROWS)],
            outp_hbm.at[c, pl.ds(s * ZROWS, ZROWS)],
        )

    return k(x, edges)


def _combine_body(a_ref, b_ref, o_ref):
    o_ref[...] = a_ref[0] + b_ref[0]


def kernel(x, edge_index, num_nodes_to):
    del num_nodes_to
    e = edge_index.shape[1]
    edges = edge_index.astype(jnp.int32)

    if e % (2 * LANES):
        npad = (2 * LANES) - e % (2 * LANES)
        pad_ar = jnp.arange(npad, dtype=jnp.int32)
        pad = jnp.stack([pad_ar % x.shape[0],
                         N_TO + pad_ar % (ACC_ROWS - N_TO)])
        edges = jnp.concatenate([edges, pad], axis=1)
    n_chunks = edges.shape[1] // LANES

    partials = _sc_partials(x, edges, n_chunks)

    rows_per_blk = 2000
    out = pl.pallas_call(
        _combine_body,
        out_shape=jax.ShapeDtypeStruct((N_TO, D), jnp.float32),
        grid=(N_TO // rows_per_blk,),
        in_specs=[
            pl.BlockSpec((1, rows_per_blk, D), lambda i: (0, i, 0)),
            pl.BlockSpec((1, rows_per_blk, D), lambda i: (1, i, 0)),
        ],
        out_specs=pl.BlockSpec((rows_per_blk, D), lambda i: (i, 0)),
    )(partials, partials)
    return out
